# Initial kernel scaffold; baseline (speedup 1.0000x reference)
import jax
import jax.numpy as jnp
from jax.experimental import pallas as pl


def kernel(pcd):
    raise NotImplementedError("write your pallas kernel here")


# speedup vs baseline: 4.7143x; 4.7143x over previous
import functools

import jax
import jax.numpy as jnp
import numpy as np
from jax import lax
from jax.experimental import pallas as pl
from jax.experimental.pallas import tpu as pltpu
from jax.experimental.pallas import tpu_sc as plsc

_PCD_RANGE = np.array([-8.0, -8.0, -2.0, 8.0, 8.0, 2.0])
_VOXEL = np.array([1.0, 1.0, 1.0])
_ANG_BINS = 12
_MAX_DIS = 15
_GRID = ((_PCD_RANGE[3:] - _PCD_RANGE[:3]) // _VOXEL + 1).astype(np.int64)
_V = int(np.prod(_GRID))

_VPAD = 1536
_B = 4
_N = 4096
_NB = 512
_VB = 512
_HPAD = 1456


def _host_consts():
    low = _PCD_RANGE[:3]
    a, b, c = np.meshgrid(
        np.arange(_GRID[0]), np.arange(_GRID[1]), np.arange(_GRID[2]), indexing="ij"
    )
    disp = np.stack([a, b, c], axis=-1).astype(np.float64) * _VOXEL
    locs = (low + disp).reshape(-1, 3).astype(np.float32)
    locs_pad = np.full((8, _VPAD), 1e9, dtype=np.float32)
    locs_pad[:3, :_V] = locs.T
    locs_pad[3:, :] = 0.0
    angs = np.array(
        [np.pi / _ANG_BINS * i - np.pi / 2 for i in range(_ANG_BINS)], dtype=np.float64
    )
    trig = np.zeros((_ANG_BINS, 32), dtype=np.float32)
    trig[:, :16] = np.cos(angs).astype(np.float32)[:, None]
    trig[:, 16:] = np.sin(angs).astype(np.float32)[:, None]
    return locs_pad, trig


_LOCS_PAD, _TRIG = _host_consts()


def _tc_body(p_ref, l_ref, o_ref):
    n_step = pl.program_id(2)
    x = p_ref[0, 0, :].reshape(_NB, 1)
    y = p_ref[0, 1, :].reshape(_NB, 1)
    z = p_ref[0, 2, :].reshape(_NB, 1)
    xv = l_ref[0, :].reshape(1, _VB)
    yv = l_ref[1, :].reshape(1, _VB)
    zv = l_ref[2, :].reshape(1, _VB)
    dx = x - xv
    dy = y - yv
    dz = z - zv
    d2 = dx * dx + dy * dy + dz * dz
    d = jnp.ceil(jnp.sqrt(d2))
    inv_n = np.float32(1.0 / _N)
    rows = []
    for i in range(_MAX_DIS):
        m = jnp.where(d <= np.float32(i + 1), inv_n, np.float32(0.0))
        rows.append(jnp.sum(m, axis=0, keepdims=True))
    rows.append(jnp.zeros((1, _VB), jnp.float32))
    res = jnp.concatenate(rows, axis=0)

    @pl.when(n_step == 0)
    def _():
        o_ref[0] = res

    @pl.when(n_step > 0)
    def _():
        o_ref[0] += res


def _tc_feature(pcd_pad):
    return pl.pallas_call(
        _tc_body,
        grid=(_B, _VPAD // _VB, _N // _NB),
        in_specs=[
            pl.BlockSpec((1, 8, _NB), lambda b, v, n: (b, 0, n)),
            pl.BlockSpec((8, _VB), lambda b, v, n: (0, v)),
        ],
        out_specs=pl.BlockSpec((1, 16, _VB), lambda b, v, n: (b, 0, v)),
        out_shape=jax.ShapeDtypeStruct((_B, 16, _VPAD), jnp.float32),
    )(pcd_pad, jnp.asarray(_LOCS_PAD))


def _floor_i32(t):
    t = jnp.clip(t, np.float32(-16000.0), np.float32(16000.0))
    i = t.astype(jnp.int32)
    f = i.astype(jnp.float32)
    return jnp.where(f > t, i - 1, i)


def _sc_hist_pair(pts_ref, trig_ref, hist_ref):
    cv = trig_ref[pl.ds(0, 16)]
    sv = trig_ref[pl.ds(16, 16)]
    ones = jnp.full((16,), np.float32(1.0 / _N), jnp.float32)

    def chunk(i, carry):
        base = i * 16
        x = pts_ref[0, pl.ds(base, 16)]
        y = pts_ref[1, pl.ds(base, 16)]
        z = pts_ref[2, pl.ds(base, 16)]
        xr = x * cv - y * sv
        yr = x * sv + y * cv
        xi = _floor_i32(xr + np.float32(8.5))
        yi = _floor_i32(yr + np.float32(8.5))
        zi = _floor_i32(z + np.float32(2.5))
        idx = zi + yi * 5 + xi * 85
        valid = (idx >= 0) & (idx < _V)
        plsc.addupdate_scatter(hist_ref, [idx], ones, mask=valid)
        return carry

    lax.fori_loop(0, _N // 16, chunk, 0)


def _sc_zero(hist_ref):
    zeros = jnp.zeros((16,), jnp.float32)
    for j in range(_HPAD // 16):
        hist_ref[pl.ds(j * 16, 16)] = zeros


def _sc_rot_hist(pcd_t, trig):
    info = plsc.get_sparse_core_info()
    nc = info.num_cores
    mesh = plsc.VectorSubcoreMesh(core_axis_name="c", subcore_axis_name="s")

    @functools.partial(
        pl.kernel,
        mesh=mesh,
        out_type=jax.ShapeDtypeStruct((48, _HPAD), jnp.float32),
        scratch_types=[
            pltpu.VMEM((3, _N), jnp.float32),
            pltpu.VMEM((32,), jnp.float32),
            pltpu.VMEM((_HPAD,), jnp.float32),
        ],
        compiler_params=pltpu.CompilerParams(needs_layout_passes=False),
    )
    def k(pcd_hbm, trig_hbm, out_hbm, pts_v, trig_v, hist_v):
        wid = lax.axis_index("s") * nc + lax.axis_index("c")
        b = lax.rem(wid, 4)
        r = lax.div(wid, 4)
        pltpu.sync_copy(pcd_hbm.at[b], pts_v)

        pltpu.sync_copy(trig_hbm.at[r], trig_v)
        _sc_zero(hist_v)
        _sc_hist_pair(pts_v, trig_v, hist_v)
        pltpu.sync_copy(hist_v, out_hbm.at[wid])

        @pl.when(wid < 16)
        def _():
            pltpu.sync_copy(trig_hbm.at[r + 8], trig_v)
            _sc_zero(hist_v)
            _sc_hist_pair(pts_v, trig_v, hist_v)
            pltpu.sync_copy(hist_v, out_hbm.at[wid + 32])

    return k(pcd_t, trig)


@jax.jit
def kernel(pcd):
    pcd_t = jnp.transpose(pcd, (0, 2, 1))
    pcd_pad = jnp.concatenate(
        [pcd_t, jnp.zeros((_B, 5, _N), jnp.float32)], axis=1
    )

    cnt = _tc_feature(pcd_pad)
    hist = _sc_rot_hist(pcd_t, jnp.asarray(_TRIG))

    feature = cnt[:, :_MAX_DIS, :_V].transpose(0, 2, 1)
    frot = hist[:, :_V].reshape(_ANG_BINS, _B, _V).transpose(1, 2, 0)
    return jnp.concatenate([feature, frot], axis=-1)


# speedup vs baseline: 8.0012x; 1.6972x over previous
import functools

import jax
import jax.numpy as jnp
import numpy as np
from jax import lax
from jax.experimental import pallas as pl
from jax.experimental.pallas import tpu as pltpu
from jax.experimental.pallas import tpu_sc as plsc

_PCD_RANGE = np.array([-8.0, -8.0, -2.0, 8.0, 8.0, 2.0])
_VOXEL = np.array([1.0, 1.0, 1.0])
_ANG_BINS = 12
_MAX_DIS = 15
_GRID = ((_PCD_RANGE[3:] - _PCD_RANGE[:3]) // _VOXEL + 1).astype(np.int64)
_V = int(np.prod(_GRID))

_VPAD = 1536
_B = 4
_N = 4096
_NB = 512
_VB = 512
_HPAD = 1456


def _host_consts():
    low = _PCD_RANGE[:3]
    a, b, c = np.meshgrid(
        np.arange(_GRID[0]), np.arange(_GRID[1]), np.arange(_GRID[2]), indexing="ij"
    )
    disp = np.stack([a, b, c], axis=-1).astype(np.float64) * _VOXEL
    locs = (low + disp).reshape(-1, 3).astype(np.float32)
    locs_pad = np.zeros((8, _VPAD), dtype=np.float32)
    locs_far = np.full((_VPAD, 3), 1e4, dtype=np.float32)
    locs_far[:_V] = locs
    locs_pad[0:3, :] = -2.0 * locs_far.T
    locs_pad[3, :] = (locs_far.astype(np.float64) ** 2).sum(-1).astype(np.float32) - 1.0
    angs = np.array(
        [np.pi / _ANG_BINS * i - np.pi / 2 for i in range(_ANG_BINS)], dtype=np.float64
    )
    trig = np.zeros((_ANG_BINS, 32), dtype=np.float32)
    trig[:, :16] = np.cos(angs).astype(np.float32)[:, None]
    trig[:, 16:] = np.sin(angs).astype(np.float32)[:, None]
    return locs_pad, trig


_LOCS_PAD, _TRIG = _host_consts()


_N_STEPS = _N // _NB
_CHUNKS = _NB // 8
_FLUSH = 15


def _tc_body(p_ref, l_ref, o_ref, h_ref):
    n_step = pl.program_id(2)

    @pl.when(n_step == 0)
    def _():
        h_ref[...] = jnp.zeros((16, 8, _VB), jnp.int32)

    p = p_ref[0]
    pn2 = jnp.sum(p * p, axis=1, keepdims=True)
    s = jax.lax.dot_general(
        p, l_ref[...], (((1,), (0,)), ((), ())),
        preferred_element_type=jnp.float32,
    )
    d2 = jnp.maximum(s + pn2, np.float32(0.0))
    d = jnp.sqrt(d2)
    ec = jnp.clip(jnp.ceil(d) - np.float32(1.0), np.float32(0.0),
                  np.float32(15.0)).astype(jnp.int32)
    sh = (ec & 7) << 2
    val = jnp.left_shift(jnp.int32(1), sh)
    vlo = jnp.where(ec < 8, val, jnp.int32(0))
    vhi = val - vlo

    a_lo = jnp.zeros((8, _VB), jnp.int32)
    a_hi = jnp.zeros((8, _VB), jnp.int32)
    pending = 0
    for c in range(_CHUNKS):
        a_lo = a_lo + jax.lax.slice(vlo, (8 * c, 0), (8 * c + 8, _VB))
        a_hi = a_hi + jax.lax.slice(vhi, (8 * c, 0), (8 * c + 8, _VB))
        pending += 1
        if pending == _FLUSH or c == _CHUNKS - 1:
            for f in range(8):
                h_ref[f] += (a_lo >> (4 * f)) & 15
            for f in range(7):
                h_ref[8 + f] += (a_hi >> (4 * f)) & 15
            a_lo = jnp.zeros((8, _VB), jnp.int32)
            a_hi = jnp.zeros((8, _VB), jnp.int32)
            pending = 0

    @pl.when(n_step == _N_STEPS - 1)
    def _():
        inv_n = np.float32(1.0 / _N)
        rows = []
        cum = jnp.zeros((1, _VB), jnp.int32)
        for k in range(_MAX_DIS):
            cum = cum + jnp.sum(h_ref[k], axis=0, keepdims=True)
            rows.append(cum.astype(jnp.float32) * inv_n)
        rows.append(jnp.zeros((1, _VB), jnp.float32))
        o_ref[0] = jnp.concatenate(rows, axis=0)


def _tc_feature(pcd_aug):
    return pl.pallas_call(
        _tc_body,
        grid=(_B, _VPAD // _VB, _N_STEPS),
        in_specs=[
            pl.BlockSpec((1, _NB, 8), lambda b, v, n: (b, n, 0)),
            pl.BlockSpec((8, _VB), lambda b, v, n: (0, v)),
        ],
        out_specs=pl.BlockSpec((1, 16, _VB), lambda b, v, n: (b, 0, v)),
        out_shape=jax.ShapeDtypeStruct((_B, 16, _VPAD), jnp.float32),
        scratch_shapes=[pltpu.VMEM((16, 8, _VB), jnp.int32)],
    )(pcd_aug, jnp.asarray(_LOCS_PAD))


def _floor_i32(t):
    t = jnp.clip(t, np.float32(-16000.0), np.float32(16000.0))
    i = t.astype(jnp.int32)
    f = i.astype(jnp.float32)
    return jnp.where(f > t, i - 1, i)


def _sc_hist_pair(pts_ref, trig_ref, hist_ref):
    cv = trig_ref[pl.ds(0, 16)]
    sv = trig_ref[pl.ds(16, 16)]
    ones = jnp.full((16,), np.float32(1.0 / _N), jnp.float32)

    def chunk(i, carry):
        base = i * 16
        x = pts_ref[0, pl.ds(base, 16)]
        y = pts_ref[1, pl.ds(base, 16)]
        z = pts_ref[2, pl.ds(base, 16)]
        xr = x * cv - y * sv
        yr = x * sv + y * cv
        xi = _floor_i32(xr + np.float32(8.5))
        yi = _floor_i32(yr + np.float32(8.5))
        zi = _floor_i32(z + np.float32(2.5))
        idx = zi + yi * 5 + xi * 85
        valid = (idx >= 0) & (idx < _V)
        plsc.addupdate_scatter(hist_ref, [idx], ones, mask=valid)
        return carry

    lax.fori_loop(0, _N // 16, chunk, 0)


def _sc_zero(hist_ref):
    zeros = jnp.zeros((16,), jnp.float32)
    for j in range(_HPAD // 16):
        hist_ref[pl.ds(j * 16, 16)] = zeros


def _sc_rot_hist(pcd_t, trig):
    info = plsc.get_sparse_core_info()
    nc = info.num_cores
    mesh = plsc.VectorSubcoreMesh(core_axis_name="c", subcore_axis_name="s")

    @functools.partial(
        pl.kernel,
        mesh=mesh,
        out_type=jax.ShapeDtypeStruct((48, _HPAD), jnp.float32),
        scratch_types=[
            pltpu.VMEM((3, _N), jnp.float32),
            pltpu.VMEM((32,), jnp.float32),
            pltpu.VMEM((_HPAD,), jnp.float32),
        ],
        compiler_params=pltpu.CompilerParams(needs_layout_passes=False),
    )
    def k(pcd_hbm, trig_hbm, out_hbm, pts_v, trig_v, hist_v):
        wid = lax.axis_index("s") * nc + lax.axis_index("c")
        b = lax.rem(wid, 4)
        r = lax.div(wid, 4)
        pltpu.sync_copy(pcd_hbm.at[b], pts_v)

        pltpu.sync_copy(trig_hbm.at[r], trig_v)
        _sc_zero(hist_v)
        _sc_hist_pair(pts_v, trig_v, hist_v)
        pltpu.sync_copy(hist_v, out_hbm.at[wid])

        @pl.when(wid < 16)
        def _():
            pltpu.sync_copy(trig_hbm.at[r + 8], trig_v)
            _sc_zero(hist_v)
            _sc_hist_pair(pts_v, trig_v, hist_v)
            pltpu.sync_copy(hist_v, out_hbm.at[wid + 32])

    return k(pcd_t, trig)


@jax.jit
def kernel(pcd):
    pcd_t = jnp.transpose(pcd, (0, 2, 1))
    ones_col = jnp.ones((_B, _N, 1), jnp.float32)
    zeros_cols = jnp.zeros((_B, _N, 4), jnp.float32)
    pcd_aug = jnp.concatenate([pcd, ones_col, zeros_cols], axis=-1)

    cnt = _tc_feature(pcd_aug)
    hist = _sc_rot_hist(pcd_t, jnp.asarray(_TRIG))

    feature = cnt[:, :_MAX_DIS, :_V].transpose(0, 2, 1)
    frot = hist[:, :_V].reshape(_ANG_BINS, _B, _V).transpose(1, 2, 0)
    return jnp.concatenate([feature, frot], axis=-1)


# speedup vs baseline: 10.7337x; 1.3415x over previous
import functools

import jax
import jax.numpy as jnp
import numpy as np
from jax import lax
from jax.experimental import pallas as pl
from jax.experimental.pallas import tpu as pltpu
from jax.experimental.pallas import tpu_sc as plsc

_PCD_RANGE = np.array([-8.0, -8.0, -2.0, 8.0, 8.0, 2.0])
_VOXEL = np.array([1.0, 1.0, 1.0])
_ANG_BINS = 12
_MAX_DIS = 15
_GRID = ((_PCD_RANGE[3:] - _PCD_RANGE[:3]) // _VOXEL + 1).astype(np.int64)
_V = int(np.prod(_GRID))

_VPAD = 1536
_B = 4
_N = 4096
_NB = 512
_VB = _VPAD
_HPAD = 1456


def _host_consts():
    low = _PCD_RANGE[:3]
    a, b, c = np.meshgrid(
        np.arange(_GRID[0]), np.arange(_GRID[1]), np.arange(_GRID[2]), indexing="ij"
    )
    disp = np.stack([a, b, c], axis=-1).astype(np.float64) * _VOXEL
    locs = (low + disp).reshape(-1, 3).astype(np.float32)
    locs_pad = np.zeros((16, _VPAD), dtype=np.float32)
    locs_far = np.full((_VPAD, 3), 1e4, dtype=np.float32)
    locs_far[:_V] = locs
    locs_pad[0:3, :] = -2.0 * locs_far.T
    locs_pad[3, :] = (locs_far.astype(np.float64) ** 2).sum(-1).astype(np.float32)
    locs_pad[8:11, :] = 1.0
    angs = np.array(
        [np.pi / _ANG_BINS * i - np.pi / 2 for i in range(_ANG_BINS)], dtype=np.float64
    )
    trig = np.zeros((_ANG_BINS, 32), dtype=np.float32)
    trig[:, :16] = np.cos(angs).astype(np.float32)[:, None]
    trig[:, 16:] = np.sin(angs).astype(np.float32)[:, None]
    return locs_pad, trig


_LOCS_PAD, _TRIG = _host_consts()


_N_STEPS = _N // _NB
_CHUNKS = _NB // 8
_FLUSH = 15
_BYTE_MASK = np.int32(0x0F0F0F0F)
_B2H_STEPS = (2, 5, _N_STEPS - 1)


def _tc_body(p_ref, l_ref, o_ref, h_ref, b_ref):
    n_step = pl.program_id(1)

    @pl.when(n_step == 0)
    def _():
        h_ref[...] = jnp.zeros((15, 8, _VB), jnp.int32)
        b_ref[...] = jnp.zeros((4, 8, _VB), jnp.int32)

    p = p_ref[0]
    paug = jnp.concatenate([p, p * p], axis=1)
    d2 = jax.lax.dot_general(
        paug, l_ref[...], (((1,), (0,)), ((), ())),
        preferred_element_type=jnp.float32,
    )
    d2 = jnp.maximum(d2, np.float32(1e-12))
    d = d2 * jax.lax.rsqrt(d2)
    ec = jnp.clip(jnp.ceil(d) - np.float32(1.0), np.float32(0.0),
                  np.float32(15.0)).astype(jnp.int32)
    sh = (ec & 7) << 2
    val = jnp.left_shift(jnp.int32(1), sh)
    vlo = jnp.where(ec < 8, val, jnp.int32(0))
    vhi = val - vlo

    a_lo = jnp.zeros((8, _VB), jnp.int32)
    a_hi = jnp.zeros((8, _VB), jnp.int32)
    pending = 0
    for c in range(_CHUNKS):
        a_lo = a_lo + jax.lax.slice(vlo, (8 * c, 0), (8 * c + 8, _VB))
        a_hi = a_hi + jax.lax.slice(vhi, (8 * c, 0), (8 * c + 8, _VB))
        pending += 1
        if pending == _FLUSH or c == _CHUNKS - 1:
            b_ref[0] += a_lo & _BYTE_MASK
            b_ref[1] += (a_lo >> 4) & _BYTE_MASK
            b_ref[2] += a_hi & _BYTE_MASK
            b_ref[3] += (a_hi >> 4) & _BYTE_MASK
            a_lo = jnp.zeros((8, _VB), jnp.int32)
            a_hi = jnp.zeros((8, _VB), jnp.int32)
            pending = 0

    @pl.when(functools.reduce(jnp.logical_or, [n_step == t for t in _B2H_STEPS]))
    def _():
        for k in range(_MAX_DIS):
            row = (2 if k >= 8 else 0) + (k & 1)
            jb = (k - 8 if k >= 8 else k) // 2
            h_ref[k] += (b_ref[row] >> (8 * jb)) & 255
        b_ref[...] = jnp.zeros((4, 8, _VB), jnp.int32)

    @pl.when(n_step == _N_STEPS - 1)
    def _():
        inv_n = np.float32(1.0 / _N)
        rows = []
        cum = jnp.zeros((1, _VB), jnp.int32)
        for k in range(_MAX_DIS):
            cum = cum + jnp.sum(h_ref[k], axis=0, keepdims=True)
            rows.append(cum.astype(jnp.float32) * inv_n)
        rows.append(jnp.zeros((1, _VB), jnp.float32))
        o_ref[0] = jnp.concatenate(rows, axis=0)


def _tc_feature(pcd_aug):
    return pl.pallas_call(
        _tc_body,
        grid=(_B, _N_STEPS),
        in_specs=[
            pl.BlockSpec((1, _NB, 8), lambda b, n: (b, n, 0)),
            pl.BlockSpec((16, _VB), lambda b, n: (0, 0)),
        ],
        out_specs=pl.BlockSpec((1, 16, _VB), lambda b, n: (b, 0, 0)),
        out_shape=jax.ShapeDtypeStruct((_B, 16, _VPAD), jnp.float32),
        scratch_shapes=[
            pltpu.VMEM((15, 8, _VB), jnp.int32),
            pltpu.VMEM((4, 8, _VB), jnp.int32),
        ],
    )(pcd_aug, jnp.asarray(_LOCS_PAD))


def _floor_i32(t):
    t = jnp.clip(t, np.float32(-16000.0), np.float32(16000.0))
    i = t.astype(jnp.int32)
    f = i.astype(jnp.float32)
    return jnp.where(f > t, i - 1, i)


def _sc_hist_pair(pts_ref, trig_ref, hist_ref):
    cv = trig_ref[pl.ds(0, 16)]
    sv = trig_ref[pl.ds(16, 16)]
    ones = jnp.full((16,), np.float32(1.0 / _N), jnp.float32)

    def chunk(i, carry):
        base = i * 16
        x = pts_ref[0, pl.ds(base, 16)]
        y = pts_ref[1, pl.ds(base, 16)]
        z = pts_ref[2, pl.ds(base, 16)]
        xr = x * cv - y * sv
        yr = x * sv + y * cv
        xi = _floor_i32(xr + np.float32(8.5))
        yi = _floor_i32(yr + np.float32(8.5))
        zi = _floor_i32(z + np.float32(2.5))
        idx = zi + yi * 5 + xi * 85
        valid = (idx >= 0) & (idx < _V)
        plsc.addupdate_scatter(hist_ref, [idx], ones, mask=valid)
        return carry

    lax.fori_loop(0, _N // 16, chunk, 0)


def _sc_zero(hist_ref):
    zeros = jnp.zeros((16,), jnp.float32)
    for j in range(_HPAD // 16):
        hist_ref[pl.ds(j * 16, 16)] = zeros


def _sc_rot_hist(pcd_t, trig):
    info = plsc.get_sparse_core_info()
    nc = info.num_cores
    mesh = plsc.VectorSubcoreMesh(core_axis_name="c", subcore_axis_name="s")

    @functools.partial(
        pl.kernel,
        mesh=mesh,
        out_type=jax.ShapeDtypeStruct((48, _HPAD), jnp.float32),
        scratch_types=[
            pltpu.VMEM((3, _N), jnp.float32),
            pltpu.VMEM((32,), jnp.float32),
            pltpu.VMEM((_HPAD,), jnp.float32),
        ],
        compiler_params=pltpu.CompilerParams(needs_layout_passes=False),
    )
    def k(pcd_hbm, trig_hbm, out_hbm, pts_v, trig_v, hist_v):
        wid = lax.axis_index("s") * nc + lax.axis_index("c")
        b = lax.rem(wid, 4)
        r = lax.div(wid, 4)
        pltpu.sync_copy(pcd_hbm.at[b], pts_v)

        pltpu.sync_copy(trig_hbm.at[r], trig_v)
        _sc_zero(hist_v)
        _sc_hist_pair(pts_v, trig_v, hist_v)
        pltpu.sync_copy(hist_v, out_hbm.at[wid])

        @pl.when(wid < 16)
        def _():
            pltpu.sync_copy(trig_hbm.at[r + 8], trig_v)
            _sc_zero(hist_v)
            _sc_hist_pair(pts_v, trig_v, hist_v)
            pltpu.sync_copy(hist_v, out_hbm.at[wid + 32])

    return k(pcd_t, trig)


@jax.jit
def kernel(pcd):
    pcd_t = jnp.transpose(pcd, (0, 2, 1))
    ones_col = jnp.ones((_B, _N, 1), jnp.float32)
    zeros_cols = jnp.zeros((_B, _N, 4), jnp.float32)
    pcd_aug = jnp.concatenate([pcd, ones_col, zeros_cols], axis=-1)

    cnt = _tc_feature(pcd_aug)
    hist = _sc_rot_hist(pcd_t, jnp.asarray(_TRIG))

    feature = cnt[:, :_MAX_DIS, :_V].transpose(0, 2, 1)
    frot = hist[:, :_V].reshape(_ANG_BINS, _B, _V).transpose(1, 2, 0)
    return jnp.concatenate([feature, frot], axis=-1)


# speedup vs baseline: 11.5308x; 1.0743x over previous
import functools

import jax
import jax.numpy as jnp
import numpy as np
from jax import lax
from jax.experimental import pallas as pl
from jax.experimental.pallas import tpu as pltpu
from jax.experimental.pallas import tpu_sc as plsc

_PCD_RANGE = np.array([-8.0, -8.0, -2.0, 8.0, 8.0, 2.0])
_VOXEL = np.array([1.0, 1.0, 1.0])
_ANG_BINS = 12
_MAX_DIS = 15
_GRID = ((_PCD_RANGE[3:] - _PCD_RANGE[:3]) // _VOXEL + 1).astype(np.int64)
_V = int(np.prod(_GRID))

_VPAD = 1536
_B = 4
_N = 4096
_NB = 512
_VB = _VPAD
_HPAD = 1456


def _host_consts():
    low = _PCD_RANGE[:3]
    a, b, c = np.meshgrid(
        np.arange(_GRID[0]), np.arange(_GRID[1]), np.arange(_GRID[2]), indexing="ij"
    )
    disp = np.stack([a, b, c], axis=-1).astype(np.float64) * _VOXEL
    locs = (low + disp).reshape(-1, 3).astype(np.float32)
    locs_pad = np.zeros((8, _VPAD), dtype=np.float32)
    locs_far = np.full((_VPAD, 3), 1e4, dtype=np.float32)
    locs_far[:_V] = locs
    locs_pad[0:3, :] = -2.0 * locs_far.T
    locs_pad[3, :] = (locs_far.astype(np.float64) ** 2).sum(-1).astype(np.float32)
    locs_pad[4:7, :] = 1.0
    angs = np.array(
        [np.pi / _ANG_BINS * i - np.pi / 2 for i in range(_ANG_BINS)], dtype=np.float64
    )
    trig = np.zeros((_ANG_BINS, 32), dtype=np.float32)
    trig[:, :16] = np.cos(angs).astype(np.float32)[:, None]
    trig[:, 16:] = np.sin(angs).astype(np.float32)[:, None]
    return locs_pad, trig


_LOCS_PAD, _TRIG = _host_consts()


_N_STEPS = _N // _NB
_CHUNKS = _NB // 8
_FLUSH = 15
_BYTE_MASK = np.int32(0x0F0F0F0F)
_B2H_STEPS = (2, 5, _N_STEPS - 1)


def _tc_body(p_ref, l_ref, o_ref, h_ref, b_ref):
    n_step = pl.program_id(1)

    @pl.when(n_step == 0)
    def _():
        h_ref[...] = jnp.zeros((15, 8, _VB), jnp.int32)
        b_ref[...] = jnp.zeros((4, 8, _VB), jnp.int32)

    p3 = p_ref[0]
    paug = jnp.concatenate(
        [p3, jnp.ones((_NB, 1), jnp.float32), p3 * p3,
         jnp.zeros((_NB, 1), jnp.float32)], axis=1)
    d2 = jax.lax.dot_general(
        paug, l_ref[...], (((1,), (0,)), ((), ())),
        preferred_element_type=jnp.float32,
    )
    d2 = jnp.maximum(d2, np.float32(1e-12))
    d = d2 * jax.lax.rsqrt(d2)
    ec = jnp.clip(d, np.float32(0.0), np.float32(15.0)).astype(jnp.int32)
    sh = (ec & 7) << 2
    val = jnp.left_shift(jnp.int32(1), sh)
    vlo = jnp.where(ec < 8, val, jnp.int32(0))
    vhi = val - vlo

    a_lo = jnp.zeros((8, _VB), jnp.int32)
    a_hi = jnp.zeros((8, _VB), jnp.int32)
    pending = 0
    for c in range(_CHUNKS):
        a_lo = a_lo + jax.lax.slice(vlo, (8 * c, 0), (8 * c + 8, _VB))
        a_hi = a_hi + jax.lax.slice(vhi, (8 * c, 0), (8 * c + 8, _VB))
        pending += 1
        if pending == _FLUSH or c == _CHUNKS - 1:
            b_ref[0] += a_lo & _BYTE_MASK
            b_ref[1] += (a_lo >> 4) & _BYTE_MASK
            b_ref[2] += a_hi & _BYTE_MASK
            b_ref[3] += (a_hi >> 4) & _BYTE_MASK
            a_lo = jnp.zeros((8, _VB), jnp.int32)
            a_hi = jnp.zeros((8, _VB), jnp.int32)
            pending = 0

    @pl.when(functools.reduce(jnp.logical_or, [n_step == t for t in _B2H_STEPS]))
    def _():
        for k in range(_MAX_DIS):
            row = (2 if k >= 8 else 0) + (k & 1)
            jb = (k - 8 if k >= 8 else k) // 2
            h_ref[k] += (b_ref[row] >> (8 * jb)) & 255
        b_ref[...] = jnp.zeros((4, 8, _VB), jnp.int32)

    @pl.when(n_step == _N_STEPS - 1)
    def _():
        inv_n = np.float32(1.0 / _N)
        rows = []
        cum = jnp.zeros((1, _VB), jnp.int32)
        for k in range(_MAX_DIS):
            cum = cum + jnp.sum(h_ref[k], axis=0, keepdims=True)
            rows.append(cum.astype(jnp.float32) * inv_n)
        rows.append(jnp.zeros((1, _VB), jnp.float32))
        o_ref[0] = jnp.concatenate(rows, axis=0)


def _tc_feature(pcd):
    return pl.pallas_call(
        _tc_body,
        grid=(_B, _N_STEPS),
        in_specs=[
            pl.BlockSpec((1, _NB, 3), lambda b, n: (b, n, 0)),
            pl.BlockSpec((8, _VB), lambda b, n: (0, 0)),
        ],
        out_specs=pl.BlockSpec((1, 16, _VB), lambda b, n: (b, 0, 0)),
        out_shape=jax.ShapeDtypeStruct((_B, 16, _VPAD), jnp.float32),
        scratch_shapes=[
            pltpu.VMEM((15, 8, _VB), jnp.int32),
            pltpu.VMEM((4, 8, _VB), jnp.int32),
        ],
    )(pcd, jnp.asarray(_LOCS_PAD))


def _floor_i32(t):
    t = jnp.clip(t, np.float32(-16000.0), np.float32(16000.0))
    i = t.astype(jnp.int32)
    f = i.astype(jnp.float32)
    return jnp.where(f > t, i - 1, i)


def _sc_hist_pair(pts_ref, trig_ref, hist_ref):
    cv = trig_ref[pl.ds(0, 16)]
    sv = trig_ref[pl.ds(16, 16)]
    ones = jnp.full((16,), np.float32(1.0 / _N), jnp.float32)
    iota3 = lax.iota(jnp.int32, 16) * 3

    def chunk(i, carry):
        xi_idx = iota3 + i * 48
        x = plsc.load_gather(pts_ref, [xi_idx])
        y = plsc.load_gather(pts_ref, [xi_idx + 1])
        z = plsc.load_gather(pts_ref, [xi_idx + 2])
        xr = x * cv - y * sv
        yr = x * sv + y * cv
        xi = _floor_i32(xr + np.float32(8.5))
        yi = _floor_i32(yr + np.float32(8.5))
        zi = _floor_i32(z + np.float32(2.5))
        idx = zi + yi * 5 + xi * 85
        valid = (idx >= 0) & (idx < _V)
        plsc.addupdate_scatter(hist_ref, [idx], ones, mask=valid)
        return carry

    lax.fori_loop(0, _N // 16, chunk, 0)


def _sc_zero(hist_ref):
    zeros = jnp.zeros((16,), jnp.float32)
    for j in range(_HPAD // 16):
        hist_ref[pl.ds(j * 16, 16)] = zeros


def _sc_rot_hist(pcd_t, trig):
    info = plsc.get_sparse_core_info()
    nc = info.num_cores
    mesh = plsc.VectorSubcoreMesh(core_axis_name="c", subcore_axis_name="s")

    @functools.partial(
        pl.kernel,
        mesh=mesh,
        out_type=jax.ShapeDtypeStruct((48, _HPAD), jnp.float32),
        scratch_types=[
            pltpu.VMEM((_N * 3,), jnp.float32),
            pltpu.VMEM((32,), jnp.float32),
            pltpu.VMEM((_HPAD,), jnp.float32),
        ],
        compiler_params=pltpu.CompilerParams(needs_layout_passes=False),
    )
    def k(pcd_hbm, trig_hbm, out_hbm, pts_v, trig_v, hist_v):
        wid = lax.axis_index("s") * nc + lax.axis_index("c")
        b = lax.rem(wid, 4)
        r = lax.div(wid, 4)
        pltpu.sync_copy(pcd_hbm.at[b], pts_v)

        pltpu.sync_copy(trig_hbm.at[r], trig_v)
        _sc_zero(hist_v)
        _sc_hist_pair(pts_v, trig_v, hist_v)
        pltpu.sync_copy(hist_v, out_hbm.at[wid])

        @pl.when(wid < 16)
        def _():
            pltpu.sync_copy(trig_hbm.at[r + 8], trig_v)
            _sc_zero(hist_v)
            _sc_hist_pair(pts_v, trig_v, hist_v)
            pltpu.sync_copy(hist_v, out_hbm.at[wid + 32])

    return k(pcd_t, trig)


@jax.jit
def kernel(pcd):
    cnt = _tc_feature(pcd)
    hist = _sc_rot_hist(pcd.reshape(_B, _N * 3), jnp.asarray(_TRIG))

    feature = cnt[:, :_MAX_DIS, :_V].transpose(0, 2, 1)
    frot = hist[:, :_V].reshape(_ANG_BINS, _B, _V).transpose(1, 2, 0)
    return jnp.concatenate([feature, frot], axis=-1)
